# baseline (speedup 1.0000x reference)
import numpy as np
import jax
import jax.numpy as jnp
from jax.experimental import pallas as pl
from jax.experimental.pallas import tpu as pltpu

_ROWS = 4096
_PIX = 512
_BLK = 256

_FOLD = np.kron(np.eye(_PIX, dtype=np.float32), np.ones((3, 1), np.float32))


def _tc_body(g_ref, p_ref, m_ref, f_ref, s_ref, c_ref, l_ref):
    i = pl.program_id(0)

    @pl.when(i == 0)
    def _init():
        s_ref[0, 0] = 0.0
        c_ref[0, 0] = 0.0
        l_ref[0, 0] = 0.0

    d = p_ref[...] - g_ref[...]
    sq = (d * d).astype(jnp.bfloat16)
    d2 = jnp.dot(sq, f_ref[...], preferred_element_type=jnp.float32)
    dist = jnp.sqrt(d2)
    mf = m_ref[...].astype(jnp.float32)
    s_ref[0, 0] += jnp.sum(dist * mf)
    c_ref[0, 0] += jnp.sum(mf)

    @pl.when(i == pl.num_programs(0) - 1)
    def _fin():
        cnt = c_ref[0, 0]
        tot = s_ref[0, 0]
        l_ref[0, 0] = jnp.where(cnt > 0.0, tot / jnp.maximum(cnt, 1.0), 0.0)


def kernel(gt_pts3d, pred_pts3d, valid_mask):
    g2 = gt_pts3d.reshape(_ROWS, _PIX * 3)
    p2 = pred_pts3d.reshape(_ROWS, _PIX * 3)
    m2 = valid_mask.reshape(_ROWS, _PIX).view(jnp.int8)
    fold = jnp.asarray(_FOLD, dtype=jnp.bfloat16)

    grid = (_ROWS // _BLK,)
    scalar_spec = pl.BlockSpec(memory_space=pltpu.SMEM)
    _, _, l = pl.pallas_call(
        _tc_body,
        grid=grid,
        in_specs=[
            pl.BlockSpec((_BLK, _PIX * 3), lambda i: (i, 0)),
            pl.BlockSpec((_BLK, _PIX * 3), lambda i: (i, 0)),
            pl.BlockSpec((_BLK, _PIX), lambda i: (i, 0)),
            pl.BlockSpec((_PIX * 3, _PIX), lambda i: (0, 0)),
        ],
        out_specs=[scalar_spec, scalar_spec, scalar_spec],
        out_shape=[
            jax.ShapeDtypeStruct((1, 1), jnp.float32),
            jax.ShapeDtypeStruct((1, 1), jnp.float32),
            jax.ShapeDtypeStruct((1, 1), jnp.float32),
        ],
    )(g2, p2, m2, fold)
    return (l[0, 0], valid_mask)


# speedup vs baseline: 5.2160x; 5.2160x over previous
import jax
import jax.numpy as jnp
from jax.experimental import pallas as pl
from jax.experimental.pallas import tpu as pltpu

_B = 8
_H = 512
_W = 512
_BH = 256
_HCH = _H // _BH


def _body(gx, gy, gz, px, py, pz, m_ref, s_ref, c_ref, l_ref):
    i = pl.program_id(0)

    @pl.when(i == 0)
    def _init():
        s_ref[0, 0] = 0.0
        c_ref[0, 0] = 0.0
        l_ref[0, 0] = 0.0

    dx = px[...] - gx[...]
    dy = py[...] - gy[...]
    dz = pz[...] - gz[...]
    d2 = dx * dx + dy * dy + dz * dz
    dist = jnp.sqrt(d2)
    mf = m_ref[...].astype(jnp.float32)
    s_ref[0, 0] += jnp.sum(dist * mf)
    c_ref[0, 0] += jnp.sum(mf)

    @pl.when(i == pl.num_programs(0) - 1)
    def _fin():
        cnt = c_ref[0, 0]
        tot = s_ref[0, 0]
        l_ref[0, 0] = jnp.where(cnt > 0.0, tot / jnp.maximum(cnt, 1.0), 0.0)


def _comp_spec(c):
    return pl.BlockSpec(
        (1, _BH, _W), lambda i, c=c: (3 * (i // _HCH) + c, i % _HCH, 0)
    )


def kernel(gt_pts3d, pred_pts3d, valid_mask):
    gp = jnp.transpose(gt_pts3d, (0, 3, 1, 2)).reshape(3 * _B, _H, _W)
    pp = jnp.transpose(pred_pts3d, (0, 3, 1, 2)).reshape(3 * _B, _H, _W)

    grid = (_B * _HCH,)
    mask_spec = pl.BlockSpec((1, _BH, _W), lambda i: (i // _HCH, i % _HCH, 0))
    scalar_spec = pl.BlockSpec(memory_space=pltpu.SMEM)
    _, _, l = pl.pallas_call(
        _body,
        grid=grid,
        in_specs=[
            _comp_spec(0), _comp_spec(1), _comp_spec(2),
            _comp_spec(0), _comp_spec(1), _comp_spec(2),
            mask_spec,
        ],
        out_specs=[scalar_spec, scalar_spec, scalar_spec],
        out_shape=[
            jax.ShapeDtypeStruct((1, 1), jnp.float32),
            jax.ShapeDtypeStruct((1, 1), jnp.float32),
            jax.ShapeDtypeStruct((1, 1), jnp.float32),
        ],
    )(gp, gp, gp, pp, pp, pp, valid_mask)
    return (l[0, 0], valid_mask)


# speedup vs baseline: 6.4353x; 1.2338x over previous
import jax
import jax.numpy as jnp
from jax.experimental import pallas as pl
from jax.experimental.pallas import tpu as pltpu

_B = 8
_H = 512
_W = 512
_BH = 512
_HCH = _H // _BH


def _body(gx, gy, gz, px, py, pz, m_ref, s_ref, c_ref, l_ref):
    i = pl.program_id(0)

    @pl.when(i == 0)
    def _init():
        s_ref[0, 0] = 0.0
        c_ref[0, 0] = 0.0
        l_ref[0, 0] = 0.0

    dx = px[...] - gx[...]
    dy = py[...] - gy[...]
    dz = pz[...] - gz[...]
    d2 = dx * dx + dy * dy + dz * dz
    dist = jnp.sqrt(d2)
    mf = (m_ref[...] != 0).astype(jnp.float32)
    s_ref[0, 0] += jnp.sum(dist * mf)
    c_ref[0, 0] += jnp.sum(mf)

    @pl.when(i == pl.num_programs(0) - 1)
    def _fin():
        cnt = c_ref[0, 0]
        tot = s_ref[0, 0]
        l_ref[0, 0] = jnp.where(cnt > 0.0, tot / jnp.maximum(cnt, 1.0), 0.0)


def _comp_spec(c):
    return pl.BlockSpec(
        (1, _BH, _W), lambda i, c=c: (3 * (i // _HCH) + c, i % _HCH, 0)
    )


def kernel(gt_pts3d, pred_pts3d, valid_mask):
    gp = jnp.transpose(gt_pts3d, (0, 3, 1, 2)).reshape(3 * _B, _H, _W)
    pp = jnp.transpose(pred_pts3d, (0, 3, 1, 2)).reshape(3 * _B, _H, _W)

    grid = (_B * _HCH,)
    mask_spec = pl.BlockSpec((1, _BH, _W), lambda i: (i // _HCH, i % _HCH, 0))
    scalar_spec = pl.BlockSpec(memory_space=pltpu.SMEM)
    _, _, l = pl.pallas_call(
        _body,
        grid=grid,
        in_specs=[
            _comp_spec(0), _comp_spec(1), _comp_spec(2),
            _comp_spec(0), _comp_spec(1), _comp_spec(2),
            mask_spec,
        ],
        out_specs=[scalar_spec, scalar_spec, scalar_spec],
        out_shape=[
            jax.ShapeDtypeStruct((1, 1), jnp.float32),
            jax.ShapeDtypeStruct((1, 1), jnp.float32),
            jax.ShapeDtypeStruct((1, 1), jnp.float32),
        ],
    )(gp, gp, gp, pp, pp, pp, valid_mask.view(jnp.int8))
    return (l[0, 0], valid_mask)


# speedup vs baseline: 6.4683x; 1.0051x over previous
import jax
import jax.numpy as jnp
from jax.experimental import pallas as pl
from jax.experimental.pallas import tpu as pltpu

_B = 8
_H = 512
_W = 512


def _body(g_ref, p_ref, m_ref, s_ref, c_ref, l_ref):
    i = pl.program_id(0)

    @pl.when(i == 0)
    def _init():
        s_ref[0, 0] = 0.0
        c_ref[0, 0] = 0.0
        l_ref[0, 0] = 0.0

    dx = p_ref[0] - g_ref[0]
    dy = p_ref[1] - g_ref[1]
    dz = p_ref[2] - g_ref[2]
    d2 = dx * dx + dy * dy + dz * dz
    dist = jnp.sqrt(d2)
    mf = (m_ref[0] != 0).astype(jnp.float32)
    s_ref[0, 0] += jnp.sum(dist * mf)
    c_ref[0, 0] += jnp.sum(mf)

    @pl.when(i == pl.num_programs(0) - 1)
    def _fin():
        cnt = c_ref[0, 0]
        tot = s_ref[0, 0]
        l_ref[0, 0] = jnp.where(cnt > 0.0, tot / jnp.maximum(cnt, 1.0), 0.0)


def kernel(gt_pts3d, pred_pts3d, valid_mask):
    gp = jnp.transpose(gt_pts3d, (0, 3, 1, 2)).reshape(3 * _B, _H, _W)
    pp = jnp.transpose(pred_pts3d, (0, 3, 1, 2)).reshape(3 * _B, _H, _W)

    grid = (_B,)
    plane_spec = pl.BlockSpec((3, _H, _W), lambda i: (i, 0, 0))
    mask_spec = pl.BlockSpec((1, _H, _W), lambda i: (i, 0, 0))
    scalar_spec = pl.BlockSpec(memory_space=pltpu.SMEM)
    _, _, l = pl.pallas_call(
        _body,
        grid=grid,
        in_specs=[plane_spec, plane_spec, mask_spec],
        out_specs=[scalar_spec, scalar_spec, scalar_spec],
        out_shape=[
            jax.ShapeDtypeStruct((1, 1), jnp.float32),
            jax.ShapeDtypeStruct((1, 1), jnp.float32),
            jax.ShapeDtypeStruct((1, 1), jnp.float32),
        ],
    )(gp, pp, valid_mask.view(jnp.int8))
    return (l[0, 0], valid_mask)
